# baseline (speedup 1.0000x reference)
import jax
import jax.numpy as jnp
from jax.experimental import pallas as pl
from jax.experimental.pallas import tpu as pltpu


def _erf(x):
    p = jnp.float32(0.3275911)
    a1 = jnp.float32(0.254829592)
    a2 = jnp.float32(-0.284496736)
    a3 = jnp.float32(1.421413741)
    a4 = jnp.float32(-1.453152027)
    a5 = jnp.float32(1.061405429)
    ax = jnp.abs(x)
    t = 1.0 / (1.0 + p * ax)
    poly = t * (a1 + t * (a2 + t * (a3 + t * (a4 + t * a5))))
    y = 1.0 - poly * jnp.exp(-ax * ax)
    return jnp.sign(x) * y


def _gelu(x):
    return x * 0.5 * (1.0 + _erf(x * jnp.float32(0.7071067811865476)))


def _fused_kernel(x_ref, ids_ref, w1_ref, b1_ref, w2_ref, b2_ref, w3_ref,
                  b3_ref, ds_ref, do_ref, o_ref, a_s, b_s, ut_s, vt_s):
    h = pl.program_id(0)
    S = x_ref.shape[0]
    E = w1_ref.shape[0]
    H = ds_ref.shape[1]

    @pl.when(h == 0)
    def _stage1():
        x = x_ref[...]
        ids = ids_ref[...]
        eiota = jax.lax.broadcasted_iota(jnp.int32, (S, E), 1)
        onehot = (ids == eiota).astype(jnp.float32)

        def dot_t(a, w):
            return jax.lax.dot_general(a, w, (((1,), (1,)), ((), ())),
                                       preferred_element_type=jnp.float32)

        pb = jnp.zeros((S, H), jnp.float32)
        for e in range(E):
            h1 = _gelu(dot_t(x, w1_ref[e]) + b1_ref[e])
            h2 = _gelu(dot_t(h1, w2_ref[e]) + b2_ref[e])
            eo = dot_t(h2, w3_ref[e]) + b3_ref[e]
            pb = pb + onehot[:, e:e + 1] * eo

        counts = jnp.sum(onehot, axis=0, keepdims=True)
        inv_s = jnp.float32(1.0 / S)
        c = jnp.dot(counts, ds_ref[...], preferred_element_type=jnp.float32) \
            * (inv_s * jnp.float32(0.01))
        off = jnp.dot(counts, do_ref[...], preferred_element_type=jnp.float32) * inv_s

        kk = jax.lax.broadcasted_iota(jnp.int32, (S, 1), 0).astype(jnp.float32) \
            - jnp.float32(S // 2)
        u = jnp.exp(-kk * c)
        v = jnp.exp(kk * c)
        pbo = pb * jnp.exp(off)
        a_s[...] = pbo * u
        b_s[...] = pbo * v

        hiota = jax.lax.broadcasted_iota(jnp.int32, (H, H), 1)
        hsel = (hiota == jax.lax.broadcasted_iota(jnp.int32, (H, H), 0))
        eyeh = hsel.astype(jnp.float32)
        cT = jnp.sum(jnp.broadcast_to(c, (H, H)) * eyeh, axis=1, keepdims=True)
        kl = jax.lax.broadcasted_iota(jnp.int32, (1, S), 1).astype(jnp.float32) \
            - jnp.float32(S // 2)
        ut_s[...] = jnp.exp(-kl * cT)
        vt_s[...] = jnp.exp(kl * cT)

    lane_h = jax.lax.broadcasted_iota(jnp.int32, (1, a_s.shape[1]), 1) == h
    onel = lane_h.astype(jnp.float32)
    a = jnp.sum(a_s[...] * onel, axis=1, keepdims=True)
    b = jnp.sum(b_s[...] * onel, axis=1, keepdims=True)
    sub_h = jax.lax.broadcasted_iota(jnp.int32, (ut_s.shape[0], 1), 0) == h
    ones = sub_h.astype(jnp.float32)
    u = jnp.sum(ut_s[...] * ones, axis=0, keepdims=True)
    v = jnp.sum(vt_s[...] * ones, axis=0, keepdims=True)
    CH = 256
    rloc = jax.lax.broadcasted_iota(jnp.int32, (CH, CH), 0)
    cloc = jax.lax.broadcasted_iota(jnp.int32, (CH, CH), 1)
    dmask = rloc >= cloc
    for i0 in range(0, S, CH):
        i1 = i0 + CH
        ac = a[i0:i1]
        bc = b[i0:i1]
        if i0 > 0:
            o_ref[0, i0:i1, 0:i0] = ac * v[:, 0:i0]
        o_ref[0, i0:i1, i0:i1] = jnp.where(dmask, ac * v[:, i0:i1],
                                           bc * u[:, i0:i1])
        if i1 < S:
            o_ref[0, i0:i1, i1:S] = bc * u[:, i1:S]


def kernel(scalars, seq_len, inscription_ids, W1, b1, W2, b2, W3, b3,
           distance_scales, distance_offsets):
    del seq_len
    B, S, D = scalars.shape
    E, HID, _ = W1.shape
    H = W3.shape[1]

    x = scalars.reshape(S, D)
    ids = inscription_ids.reshape(S, 1).astype(jnp.int32)

    def whole(shape):
        return pl.BlockSpec(shape, lambda h: (0,) * len(shape))

    bias = pl.pallas_call(
        _fused_kernel,
        grid=(H,),
        in_specs=[
            whole((S, D)), whole((S, 1)),
            whole((E, HID, D)), whole((E, HID)),
            whole((E, HID, HID)), whole((E, HID)),
            whole((E, H, HID)), whole((E, H)),
            whole((E, H)), whole((E, H)),
        ],
        out_specs=pl.BlockSpec((1, S, S), lambda h: (h, 0, 0)),
        out_shape=jax.ShapeDtypeStruct((H, S, S), jnp.float32),
        scratch_shapes=[
            pltpu.VMEM((S, H), jnp.float32),
            pltpu.VMEM((S, H), jnp.float32),
            pltpu.VMEM((H, S), jnp.float32),
            pltpu.VMEM((H, S), jnp.float32),
        ],
    )(x, ids, W1, b1, W2, b2, W3, b3, distance_scales, distance_offsets)

    return bias.reshape(B, H, S, S)


# speedup vs baseline: 1.1416x; 1.1416x over previous
import jax
import jax.numpy as jnp
from jax.experimental import pallas as pl
from jax.experimental.pallas import tpu as pltpu


def _erf(x):
    p = jnp.float32(0.3275911)
    a1 = jnp.float32(0.254829592)
    a2 = jnp.float32(-0.284496736)
    a3 = jnp.float32(1.421413741)
    a4 = jnp.float32(-1.453152027)
    a5 = jnp.float32(1.061405429)
    ax = jnp.abs(x)
    t = 1.0 / (1.0 + p * ax)
    poly = t * (a1 + t * (a2 + t * (a3 + t * (a4 + t * a5))))
    y = 1.0 - poly * jnp.exp(-ax * ax)
    return jnp.sign(x) * y


def _gelu(x):
    return x * 0.5 * (1.0 + _erf(x * jnp.float32(0.7071067811865476)))


def _fused_kernel(x_ref, ids_ref, w1_ref, b1_ref, w2_ref, b2_ref, w3_ref,
                  b3_ref, ds_ref, do_ref, o_ref, a_s, b_s, ut_s, vt_s):
    h = pl.program_id(0)
    S = x_ref.shape[0]
    E = w1_ref.shape[0]
    H = ds_ref.shape[1]

    @pl.when(h == 0)
    def _stage1():
        x = x_ref[...]
        ids = ids_ref[...]
        eiota = jax.lax.broadcasted_iota(jnp.int32, (S, E), 1)
        onehot = (ids == eiota).astype(jnp.float32)

        def dot_t(a, w):
            return jax.lax.dot_general(a, w, (((1,), (1,)), ((), ())),
                                       preferred_element_type=jnp.float32)

        def dot_n(a, w):
            return jax.lax.dot_general(a, w, (((1,), (0,)), ((), ())),
                                       preferred_element_type=jnp.float32)

        def routed_layer(inp, w_ref, b_ref):
            z = dot_n(onehot, b_ref[...])
            for e in range(E):
                z = z + onehot[:, e:e + 1] * dot_t(inp, w_ref[e])
            return z

        h1 = _gelu(routed_layer(x, w1_ref, b1_ref))
        h2 = _gelu(routed_layer(h1, w2_ref, b2_ref))
        pb = routed_layer(h2, w3_ref, b3_ref)

        counts = jnp.sum(onehot, axis=0, keepdims=True)
        inv_s = jnp.float32(1.0 / S)
        c = jnp.dot(counts, ds_ref[...], preferred_element_type=jnp.float32) \
            * (inv_s * jnp.float32(0.01))
        off = jnp.dot(counts, do_ref[...], preferred_element_type=jnp.float32) * inv_s

        kk = jax.lax.broadcasted_iota(jnp.int32, (S, 1), 0).astype(jnp.float32) \
            - jnp.float32(S // 2)
        u = jnp.exp(-kk * c)
        v = jnp.exp(kk * c)
        pbo = pb * jnp.exp(off)
        a_s[...] = pbo * u
        b_s[...] = pbo * v

        hiota = jax.lax.broadcasted_iota(jnp.int32, (H, H), 1)
        hsel = (hiota == jax.lax.broadcasted_iota(jnp.int32, (H, H), 0))
        eyeh = hsel.astype(jnp.float32)
        cT = jnp.sum(jnp.broadcast_to(c, (H, H)) * eyeh, axis=1, keepdims=True)
        kl = jax.lax.broadcasted_iota(jnp.int32, (1, S), 1).astype(jnp.float32) \
            - jnp.float32(S // 2)
        ut_s[...] = jnp.exp(-kl * cT)
        vt_s[...] = jnp.exp(kl * cT)

    lane_h = jax.lax.broadcasted_iota(jnp.int32, (1, a_s.shape[1]), 1) == h
    onel = lane_h.astype(jnp.float32)
    a = jnp.sum(a_s[...] * onel, axis=1, keepdims=True)
    b = jnp.sum(b_s[...] * onel, axis=1, keepdims=True)
    sub_h = jax.lax.broadcasted_iota(jnp.int32, (ut_s.shape[0], 1), 0) == h
    ones = sub_h.astype(jnp.float32)
    u = jnp.sum(ut_s[...] * ones, axis=0, keepdims=True)
    v = jnp.sum(vt_s[...] * ones, axis=0, keepdims=True)
    CH = 256
    rloc = jax.lax.broadcasted_iota(jnp.int32, (CH, CH), 0)
    cloc = jax.lax.broadcasted_iota(jnp.int32, (CH, CH), 1)
    dmask = rloc >= cloc
    for i0 in range(0, S, CH):
        i1 = i0 + CH
        ac = a[i0:i1]
        bc = b[i0:i1]
        if i0 > 0:
            o_ref[0, i0:i1, 0:i0] = ac * v[:, 0:i0]
        o_ref[0, i0:i1, i0:i1] = jnp.where(dmask, ac * v[:, i0:i1],
                                           bc * u[:, i0:i1])
        if i1 < S:
            o_ref[0, i0:i1, i1:S] = bc * u[:, i1:S]


def kernel(scalars, seq_len, inscription_ids, W1, b1, W2, b2, W3, b3,
           distance_scales, distance_offsets):
    del seq_len
    B, S, D = scalars.shape
    E, HID, _ = W1.shape
    H = W3.shape[1]

    x = scalars.reshape(S, D)
    ids = inscription_ids.reshape(S, 1).astype(jnp.int32)

    def whole(shape):
        return pl.BlockSpec(shape, lambda h: (0,) * len(shape))

    bias = pl.pallas_call(
        _fused_kernel,
        grid=(H,),
        in_specs=[
            whole((S, D)), whole((S, 1)),
            whole((E, HID, D)), whole((E, HID)),
            whole((E, HID, HID)), whole((E, HID)),
            whole((E, H, HID)), whole((E, H)),
            whole((E, H)), whole((E, H)),
        ],
        out_specs=pl.BlockSpec((1, S, S), lambda h: (h, 0, 0)),
        out_shape=jax.ShapeDtypeStruct((H, S, S), jnp.float32),
        scratch_shapes=[
            pltpu.VMEM((S, H), jnp.float32),
            pltpu.VMEM((S, H), jnp.float32),
            pltpu.VMEM((H, S), jnp.float32),
            pltpu.VMEM((H, S), jnp.float32),
        ],
    )(x, ids, W1, b1, W2, b2, W3, b3, distance_scales, distance_offsets)

    return bias.reshape(B, H, S, S)


# speedup vs baseline: 1.1487x; 1.0062x over previous
import jax
import jax.numpy as jnp
from jax.experimental import pallas as pl
from jax.experimental.pallas import tpu as pltpu


def _erf(x):
    p = jnp.float32(0.3275911)
    a1 = jnp.float32(0.254829592)
    a2 = jnp.float32(-0.284496736)
    a3 = jnp.float32(1.421413741)
    a4 = jnp.float32(-1.453152027)
    a5 = jnp.float32(1.061405429)
    ax = jnp.abs(x)
    t = 1.0 / (1.0 + p * ax)
    poly = t * (a1 + t * (a2 + t * (a3 + t * (a4 + t * a5))))
    y = 1.0 - poly * jnp.exp(-ax * ax)
    return jnp.sign(x) * y


def _gelu(x):
    return x * 0.5 * (1.0 + _erf(x * jnp.float32(0.7071067811865476)))


def _fused_kernel(x_ref, ids_ref, w1_ref, b1_ref, w2_ref, b2_ref, w3_ref,
                  b3_ref, ds_ref, do_ref, o_ref, a_s, b_s, ut_s, vt_s,
                  buf0, buf1, buf2, buf3, sem0, sem1, sem2, sem3):
    bufs = (buf0, buf1, buf2, buf3)
    sems = (sem0, sem1, sem2, sem3)
    h = pl.program_id(0)
    S = x_ref.shape[0]
    E = w1_ref.shape[0]
    H = ds_ref.shape[1]

    @pl.when(h == 0)
    def _stage1():
        x = x_ref[...]
        ids = ids_ref[...]
        eiota = jax.lax.broadcasted_iota(jnp.int32, (S, E), 1)
        onehot = (ids == eiota).astype(jnp.float32)

        def dot_t(a, w):
            return jax.lax.dot_general(a, w, (((1,), (1,)), ((), ())),
                                       preferred_element_type=jnp.float32)

        def dot_n(a, w):
            return jax.lax.dot_general(a, w, (((1,), (0,)), ((), ())),
                                       preferred_element_type=jnp.float32)

        def routed_layer(inp, w_ref, b_ref):
            z = dot_n(onehot, b_ref[...])
            for e in range(E):
                z = z + onehot[:, e:e + 1] * dot_t(inp, w_ref[e])
            return z

        h1 = _gelu(routed_layer(x, w1_ref, b1_ref))
        h2 = _gelu(routed_layer(h1, w2_ref, b2_ref))
        pb = routed_layer(h2, w3_ref, b3_ref)

        counts = jnp.sum(onehot, axis=0, keepdims=True)
        inv_s = jnp.float32(1.0 / S)
        c = jnp.dot(counts, ds_ref[...], preferred_element_type=jnp.float32) \
            * (inv_s * jnp.float32(0.01))
        off = jnp.dot(counts, do_ref[...], preferred_element_type=jnp.float32) * inv_s

        kk = jax.lax.broadcasted_iota(jnp.int32, (S, 1), 0).astype(jnp.float32) \
            - jnp.float32(S // 2)
        u = jnp.exp(-kk * c)
        v = jnp.exp(kk * c)
        pbo = pb * jnp.exp(off)
        a_s[...] = pbo * u
        b_s[...] = pbo * v

        hiota = jax.lax.broadcasted_iota(jnp.int32, (H, H), 1)
        hsel = (hiota == jax.lax.broadcasted_iota(jnp.int32, (H, H), 0))
        eyeh = hsel.astype(jnp.float32)
        cT = jnp.sum(jnp.broadcast_to(c, (H, H)) * eyeh, axis=1, keepdims=True)
        kl = jax.lax.broadcasted_iota(jnp.int32, (1, S), 1).astype(jnp.float32) \
            - jnp.float32(S // 2)
        ut_s[...] = jnp.exp(-kl * cT)
        vt_s[...] = jnp.exp(kl * cT)

    lane_h = jax.lax.broadcasted_iota(jnp.int32, (1, a_s.shape[1]), 1) == h
    onel = lane_h.astype(jnp.float32)
    a = jnp.sum(a_s[...] * onel, axis=1, keepdims=True)
    b = jnp.sum(b_s[...] * onel, axis=1, keepdims=True)
    sub_h = jax.lax.broadcasted_iota(jnp.int32, (ut_s.shape[0], 1), 0) == h
    ones = sub_h.astype(jnp.float32)
    u = jnp.sum(ut_s[...] * ones, axis=0, keepdims=True)
    v = jnp.sum(vt_s[...] * ones, axis=0, keepdims=True)
    CH = 256
    rloc = jax.lax.broadcasted_iota(jnp.int32, (CH, CH), 0)
    cloc = jax.lax.broadcasted_iota(jnp.int32, (CH, CH), 1)
    dmask = rloc >= cloc
    nbuf = len(bufs)
    chr_ = bufs[0].shape[0]
    H_total = o_ref.shape[0]
    for k in range(nbuf):
        copy = pltpu.make_async_copy(
            bufs[k], o_ref.at[h, pl.ds(k * chr_, chr_), :], sems[k])

        @pl.when(h > 0)
        def _wait_prev(copy=copy):
            copy.wait()

        for r0 in range(0, chr_, CH):
            i0 = k * chr_ + r0
            i1 = i0 + CH
            ac = a[i0:i1]
            bc = b[i0:i1]
            if i0 > 0:
                bufs[k][r0:r0 + CH, 0:i0] = ac * v[:, 0:i0]
            bufs[k][r0:r0 + CH, i0:i1] = jnp.where(dmask, ac * v[:, i0:i1],
                                                   bc * u[:, i0:i1])
            if i1 < S:
                bufs[k][r0:r0 + CH, i1:S] = bc * u[:, i1:S]
        copy.start()

    @pl.when(h == H_total - 1)
    def _drain():
        for k in range(nbuf):
            pltpu.make_async_copy(
                bufs[k], o_ref.at[h, pl.ds(k * chr_, chr_), :], sems[k]).wait()


def kernel(scalars, seq_len, inscription_ids, W1, b1, W2, b2, W3, b3,
           distance_scales, distance_offsets):
    del seq_len
    B, S, D = scalars.shape
    E, HID, _ = W1.shape
    H = W3.shape[1]

    x = scalars.reshape(S, D)
    ids = inscription_ids.reshape(S, 1).astype(jnp.int32)

    def whole(shape):
        return pl.BlockSpec(shape, lambda h: (0,) * len(shape))

    bias = pl.pallas_call(
        _fused_kernel,
        grid=(H,),
        in_specs=[
            whole((S, D)), whole((S, 1)),
            whole((E, HID, D)), whole((E, HID)),
            whole((E, HID, HID)), whole((E, HID)),
            whole((E, H, HID)), whole((E, H)),
            whole((E, H)), whole((E, H)),
        ],
        out_specs=pl.BlockSpec(memory_space=pltpu.MemorySpace.HBM),
        out_shape=jax.ShapeDtypeStruct((H, S, S), jnp.float32),
        scratch_shapes=[
            pltpu.VMEM((S, H), jnp.float32),
            pltpu.VMEM((S, H), jnp.float32),
            pltpu.VMEM((H, S), jnp.float32),
            pltpu.VMEM((H, S), jnp.float32),
            pltpu.VMEM((S // 4, S), jnp.float32),
            pltpu.VMEM((S // 4, S), jnp.float32),
            pltpu.VMEM((S // 4, S), jnp.float32),
            pltpu.VMEM((S // 4, S), jnp.float32),
            pltpu.SemaphoreType.DMA,
            pltpu.SemaphoreType.DMA,
            pltpu.SemaphoreType.DMA,
            pltpu.SemaphoreType.DMA,
        ],
    )(x, ids, W1, b1, W2, b2, W3, b3, distance_scales, distance_offsets)

    return bias.reshape(B, H, S, S)


# speedup vs baseline: 1.1526x; 1.0034x over previous
import jax
import jax.numpy as jnp
from jax.experimental import pallas as pl
from jax.experimental.pallas import tpu as pltpu


def _erf(x):
    p = jnp.float32(0.3275911)
    a1 = jnp.float32(0.254829592)
    a2 = jnp.float32(-0.284496736)
    a3 = jnp.float32(1.421413741)
    a4 = jnp.float32(-1.453152027)
    a5 = jnp.float32(1.061405429)
    ax = jnp.abs(x)
    t = 1.0 / (1.0 + p * ax)
    poly = t * (a1 + t * (a2 + t * (a3 + t * (a4 + t * a5))))
    y = 1.0 - poly * jnp.exp(-ax * ax)
    return jnp.sign(x) * y


def _gelu(x):
    return x * 0.5 * (1.0 + _erf(x * jnp.float32(0.7071067811865476)))


def _fused_kernel(x_ref, ids_ref, w1_ref, b1_ref, w2_ref, b2_ref, w3_ref,
                  b3_ref, ds_ref, do_ref, o_ref, a_s, b_s, ut_s, vt_s,
                  buf0, buf1, buf2, buf3, buf4, buf5, buf6, buf7,
                  sem0, sem1, sem2, sem3, sem4, sem5, sem6, sem7):
    bufs = (buf0, buf1, buf2, buf3, buf4, buf5, buf6, buf7)
    sems = (sem0, sem1, sem2, sem3, sem4, sem5, sem6, sem7)
    h = pl.program_id(0)
    S = x_ref.shape[0]
    E = w1_ref.shape[0]
    H = ds_ref.shape[1]

    @pl.when(h == 0)
    def _stage1():
        x = x_ref[...]
        ids = ids_ref[...]
        eiota = jax.lax.broadcasted_iota(jnp.int32, (S, E), 1)
        onehot = (ids == eiota).astype(jnp.float32)

        def dot_t(a, w):
            return jax.lax.dot_general(a, w, (((1,), (1,)), ((), ())),
                                       preferred_element_type=jnp.float32)

        def dot_n(a, w):
            return jax.lax.dot_general(a, w, (((1,), (0,)), ((), ())),
                                       preferred_element_type=jnp.float32)

        def routed_layer(inp, w_ref, b_ref):
            z = dot_n(onehot, b_ref[...])
            for e in range(E):
                z = z + onehot[:, e:e + 1] * dot_t(inp, w_ref[e])
            return z

        h1 = _gelu(routed_layer(x, w1_ref, b1_ref))
        h2 = _gelu(routed_layer(h1, w2_ref, b2_ref))
        pb = routed_layer(h2, w3_ref, b3_ref)

        counts = jnp.sum(onehot, axis=0, keepdims=True)
        inv_s = jnp.float32(1.0 / S)
        c = jnp.dot(counts, ds_ref[...], preferred_element_type=jnp.float32) \
            * (inv_s * jnp.float32(0.01))
        off = jnp.dot(counts, do_ref[...], preferred_element_type=jnp.float32) * inv_s

        kk = jax.lax.broadcasted_iota(jnp.int32, (S, 1), 0).astype(jnp.float32) \
            - jnp.float32(S // 2)
        u = jnp.exp(-kk * c)
        v = jnp.exp(kk * c)
        pbo = pb * jnp.exp(off)
        a_s[...] = pbo * u
        b_s[...] = pbo * v

        hiota = jax.lax.broadcasted_iota(jnp.int32, (H, H), 1)
        hsel = (hiota == jax.lax.broadcasted_iota(jnp.int32, (H, H), 0))
        eyeh = hsel.astype(jnp.float32)
        cT = jnp.sum(jnp.broadcast_to(c, (H, H)) * eyeh, axis=1, keepdims=True)
        kl = jax.lax.broadcasted_iota(jnp.int32, (1, S), 1).astype(jnp.float32) \
            - jnp.float32(S // 2)
        ut_s[...] = jnp.exp(-kl * cT)
        vt_s[...] = jnp.exp(kl * cT)

    lane_h = jax.lax.broadcasted_iota(jnp.int32, (1, a_s.shape[1]), 1) == h
    onel = lane_h.astype(jnp.float32)
    a = jnp.sum(a_s[...] * onel, axis=1, keepdims=True)
    b = jnp.sum(b_s[...] * onel, axis=1, keepdims=True)
    sub_h = jax.lax.broadcasted_iota(jnp.int32, (ut_s.shape[0], 1), 0) == h
    ones = sub_h.astype(jnp.float32)
    u = jnp.sum(ut_s[...] * ones, axis=0, keepdims=True)
    v = jnp.sum(vt_s[...] * ones, axis=0, keepdims=True)
    CH = 256
    rloc = jax.lax.broadcasted_iota(jnp.int32, (CH, CH), 0)
    cloc = jax.lax.broadcasted_iota(jnp.int32, (CH, CH), 1)
    dmask = rloc >= cloc
    nbuf = len(bufs)
    chr_ = bufs[0].shape[0]
    H_total = o_ref.shape[0]
    for k in range(nbuf):
        copy = pltpu.make_async_copy(
            bufs[k], o_ref.at[h, pl.ds(k * chr_, chr_), :], sems[k])

        @pl.when(h > 0)
        def _wait_prev(copy=copy):
            copy.wait()

        for r0 in range(0, chr_, CH):
            i0 = k * chr_ + r0
            i1 = i0 + CH
            ac = a[i0:i1]
            bc = b[i0:i1]
            if i0 > 0:
                bufs[k][r0:r0 + CH, 0:i0] = ac * v[:, 0:i0]
            bufs[k][r0:r0 + CH, i0:i1] = jnp.where(dmask, ac * v[:, i0:i1],
                                                   bc * u[:, i0:i1])
            if i1 < S:
                bufs[k][r0:r0 + CH, i1:S] = bc * u[:, i1:S]
        copy.start()

    @pl.when(h == H_total - 1)
    def _drain():
        for k in range(nbuf):
            pltpu.make_async_copy(
                bufs[k], o_ref.at[h, pl.ds(k * chr_, chr_), :], sems[k]).wait()


def kernel(scalars, seq_len, inscription_ids, W1, b1, W2, b2, W3, b3,
           distance_scales, distance_offsets):
    del seq_len
    B, S, D = scalars.shape
    E, HID, _ = W1.shape
    H = W3.shape[1]

    x = scalars.reshape(S, D)
    ids = inscription_ids.reshape(S, 1).astype(jnp.int32)

    def whole(shape):
        return pl.BlockSpec(shape, lambda h: (0,) * len(shape))

    bias = pl.pallas_call(
        _fused_kernel,
        grid=(H,),
        in_specs=[
            whole((S, D)), whole((S, 1)),
            whole((E, HID, D)), whole((E, HID)),
            whole((E, HID, HID)), whole((E, HID)),
            whole((E, H, HID)), whole((E, H)),
            whole((E, H)), whole((E, H)),
        ],
        out_specs=pl.BlockSpec(memory_space=pltpu.MemorySpace.HBM),
        out_shape=jax.ShapeDtypeStruct((H, S, S), jnp.float32),
        scratch_shapes=[
            pltpu.VMEM((S, H), jnp.float32),
            pltpu.VMEM((S, H), jnp.float32),
            pltpu.VMEM((H, S), jnp.float32),
            pltpu.VMEM((H, S), jnp.float32),
        ] + [pltpu.VMEM((S // 8, S), jnp.float32)] * 8
          + [pltpu.SemaphoreType.DMA] * 8,
    )(x, ids, W1, b1, W2, b2, W3, b3, distance_scales, distance_offsets)

    return bias.reshape(B, H, S, S)
